# baseline (speedup 1.0000x reference)
import functools

import jax
import jax.numpy as jnp
from jax.experimental import pallas as pl
from jax.experimental.pallas import tpu as pltpu


def _edge_body(row_ref, col_ref, x_ref, wl_ref, wa_ref, out_ref, xp_ref, *, block_e):
    i = pl.program_id(0)

    @pl.when(i == 0)
    def _init():
        xp_ref[...] = jax.lax.dot_general(
            x_ref[...], wl_ref[...], (((1,), (1,)), ((), ())),
            preferred_element_type=jnp.float32)
        out_ref[...] = jnp.zeros_like(out_ref)

    wa = wa_ref[...]

    def edge(e, carry):
        r = row_ref[0, 0, e]
        c = col_ref[0, 0, e]
        a = xp_ref[pl.ds(r, 1), :]
        b = xp_ref[pl.ds(c, 1), :]
        t = jnp.maximum(0.5 * a + b, 0.0)
        logits = jnp.sum(t * wa, axis=1, keepdims=True)
        mx = jnp.max(logits, axis=0, keepdims=True)
        ex = jnp.exp(logits - mx)
        p = ex / jnp.sum(ex, axis=0, keepdims=True)
        y = p * b
        out_ref[pl.ds(r, 1), :, :] += y[None, :, :]
        return carry

    jax.lax.fori_loop(0, block_e, edge, 0)


def kernel(x, edge_index, W_lin, W_att):
    n_nodes, in_feat = x.shape
    out_feat, _ = W_lin.shape
    c_dim = W_att.shape[0]
    n_edges = edge_index.shape[1]

    block_e = 2000 if n_edges % 2000 == 0 else n_edges
    nb = n_edges // block_e

    row = edge_index[0].astype(jnp.int32).reshape(nb, 1, block_e)
    col = edge_index[1].astype(jnp.int32).reshape(nb, 1, block_e)

    out = pl.pallas_call(
        functools.partial(_edge_body, block_e=block_e),
        grid=(nb,),
        in_specs=[
            pl.BlockSpec((1, 1, block_e), lambda i: (i, 0, 0),
                         memory_space=pltpu.SMEM),
            pl.BlockSpec((1, 1, block_e), lambda i: (i, 0, 0),
                         memory_space=pltpu.SMEM),
            pl.BlockSpec((n_nodes, in_feat), lambda i: (0, 0)),
            pl.BlockSpec((out_feat, in_feat), lambda i: (0, 0)),
            pl.BlockSpec((c_dim, out_feat), lambda i: (0, 0)),
        ],
        out_specs=pl.BlockSpec((n_nodes, c_dim, out_feat), lambda i: (0, 0, 0)),
        out_shape=jax.ShapeDtypeStruct((n_nodes, c_dim, out_feat), jnp.float32),
        scratch_shapes=[pltpu.VMEM((n_nodes, out_feat), jnp.float32)],
    )(row, col, x, W_lin, W_att)
    return out.reshape(n_nodes, c_dim * out_feat)


# speedup vs baseline: 16.2841x; 16.2841x over previous
import functools

import jax
import jax.numpy as jnp
from jax import lax
from jax.experimental import pallas as pl
from jax.experimental.pallas import tpu as pltpu
from jax.experimental.pallas import tpu_sc as plsc

NC = 2
NS = 16
L = 16
K = 80


def _mm_body(x_ref, wl_ref, o_ref):
    o_ref[...] = lax.dot_general(
        x_ref[...], wl_ref[...], (((1,), (1,)), ((), ())),
        preferred_element_type=jnp.float32)


def _att_body(s_ref, wa_ref, w_ref):
    lg = lax.dot_general(
        wa_ref[...], s_ref[...], (((1,), (1,)), ((), ())),
        preferred_element_type=jnp.float32)
    m = jnp.max(lg, axis=0, keepdims=True)
    ex = jnp.exp(lg - m)
    w_ref[...] = ex / jnp.sum(ex, axis=0, keepdims=True)


def _sc_relu_body(n_edges, f_dim,
                  xp_hbm, row_hbm, col_hbm, s_hbm,
                  ridx_v, cidx_v, a_v, b_v, sem_a, sem_b):
    nw = NC * NS
    epw = n_edges // nw
    nchunks = epw // K
    nj = f_dim // L

    cid = lax.axis_index("c")
    sid = lax.axis_index("s")
    base = (sid * NC + cid) * epw

    def chunk_body(t, carry):
        cb = base + t * K
        pltpu.sync_copy(row_hbm.at[pl.ds(cb, K)], ridx_v)
        pltpu.sync_copy(col_hbm.at[pl.ds(cb, K)], cidx_v)
        cp_a = pltpu.async_copy(xp_hbm.at[ridx_v], a_v, sem_a)
        cp_b = pltpu.async_copy(xp_hbm.at[cidx_v], b_v, sem_b)
        cp_a.wait()
        cp_b.wait()

        def edge_body(e, ecarry):
            for j in range(nj):
                av = a_v[e, pl.ds(j * L, L)]
                bv = b_v[e, pl.ds(j * L, L)]
                a_v[e, pl.ds(j * L, L)] = jnp.maximum(0.5 * av + bv, 0.0)
            return ecarry

        lax.fori_loop(0, K, edge_body, 0)
        pltpu.sync_copy(a_v, s_hbm.at[pl.ds(cb, K)])
        return carry

    lax.fori_loop(0, nchunks, chunk_body, 0)


def _sc_scatter_body(n_nodes, n_edges, c_dim, f_dim,
                     xp_hbm, row_hbm, col_hbm, w_hbm, z_hbm, out_hbm,
                     acc_sh, ridx_v, cidx_v, b_v, y_v, w_v, zb_v, ob_v, sem_b):
    eps = n_edges // NS
    nchunks = eps // K
    zrows = z_hbm.shape[0]
    stripe = zrows * 8
    nj = f_dim // L

    cid = lax.axis_index("c")
    sid = lax.axis_index("s")

    rbase = sid * stripe
    my_rows = jnp.clip(n_nodes - rbase, 0, stripe)
    nz = my_rows // zrows

    pltpu.sync_copy(z_hbm, zb_v)

    for cpass in range(c_dim // NC):
        c_val = cid * (c_dim // NC) + cpass

        def zero_body(r, carry):
            pltpu.sync_copy(zb_v, acc_sh.at[pl.ds(rbase + r * zrows, zrows)])
            return carry

        lax.fori_loop(0, nz, zero_body, 0)
        plsc.subcore_barrier()

        def chunk_body(t, carry):
            cb = sid * eps + t * K
            pltpu.sync_copy(row_hbm.at[pl.ds(cb, K)], ridx_v)
            pltpu.sync_copy(col_hbm.at[pl.ds(cb, K)], cidx_v)
            cp_b = pltpu.async_copy(xp_hbm.at[cidx_v], b_v, sem_b)
            pltpu.sync_copy(w_hbm.at[pl.ds(c_val * n_edges + cb, K)], w_v)
            cp_b.wait()

            def edge_body(e, ecarry):
                ws = jnp.broadcast_to(w_v[pl.ds(e, 1)], (L,))
                for j in range(nj):
                    y_v[e, pl.ds(j * L, L)] = b_v[e, pl.ds(j * L, L)] * ws
                return ecarry

            lax.fori_loop(0, K, edge_body, 0)
            pltpu.sync_copy(y_v, acc_sh.at[ridx_v], add=True)
            return carry

        lax.fori_loop(0, nchunks, chunk_body, 0)
        plsc.subcore_barrier()

        def wb_body(r, carry):
            rb = rbase + r * zrows
            pltpu.sync_copy(acc_sh.at[pl.ds(rb, zrows)], ob_v)
            pltpu.sync_copy(ob_v, out_hbm.at[pl.ds(c_val * n_nodes + rb, zrows)])
            return carry

        lax.fori_loop(0, nz, wb_body, 0)
        plsc.subcore_barrier()


def kernel(x, edge_index, W_lin, W_att):
    n_nodes, in_feat = x.shape
    f_dim = W_lin.shape[0]
    c_dim = W_att.shape[0]
    n_edges = edge_index.shape[1]
    nw = NC * NS
    assert n_edges % (nw * K) == 0 and f_dim % L == 0
    assert c_dim % NC == 0

    row = edge_index[0].astype(jnp.int32)
    col = edge_index[1].astype(jnp.int32)

    xp = pl.pallas_call(
        _mm_body,
        out_shape=jax.ShapeDtypeStruct((n_nodes, f_dim), jnp.float32),
    )(x, W_lin)

    mesh = plsc.VectorSubcoreMesh(core_axis_name="c", subcore_axis_name="s")

    s = pl.kernel(
        functools.partial(_sc_relu_body, n_edges, f_dim),
        out_type=jax.ShapeDtypeStruct((n_edges, f_dim), jnp.float32),
        mesh=mesh,
        scratch_types=[
            pltpu.VMEM((K,), jnp.int32),
            pltpu.VMEM((K,), jnp.int32),
            pltpu.VMEM((K, f_dim), jnp.float32),
            pltpu.VMEM((K, f_dim), jnp.float32),
            pltpu.SemaphoreType.DMA,
            pltpu.SemaphoreType.DMA,
        ],
    )(xp, row, col)

    be = 2560
    assert n_edges % be == 0
    w = pl.pallas_call(
        _att_body,
        grid=(n_edges // be,),
        in_specs=[
            pl.BlockSpec((be, f_dim), lambda i: (i, 0)),
            pl.BlockSpec((c_dim, f_dim), lambda i: (0, 0)),
        ],
        out_specs=pl.BlockSpec((c_dim, be), lambda i: (0, i)),
        out_shape=jax.ShapeDtypeStruct((c_dim, n_edges), jnp.float32),
    )(s, W_att)
    w_flat = w.reshape(c_dim * n_edges)

    zrows = 80
    assert n_nodes % zrows == 0 and n_nodes <= NS * zrows * 8
    z = jnp.zeros((zrows, f_dim), jnp.float32)
    out2 = pl.kernel(
        functools.partial(_sc_scatter_body, n_nodes, n_edges, c_dim, f_dim),
        out_type=jax.ShapeDtypeStruct((c_dim * n_nodes, f_dim), jnp.float32),
        mesh=mesh,
        scratch_types=[
            pltpu.VMEM_SHARED((n_nodes, f_dim), jnp.float32),
            pltpu.VMEM((K,), jnp.int32),
            pltpu.VMEM((K,), jnp.int32),
            pltpu.VMEM((K, f_dim), jnp.float32),
            pltpu.VMEM((K, f_dim), jnp.float32),
            pltpu.VMEM((K,), jnp.float32),
            pltpu.VMEM((zrows, f_dim), jnp.float32),
            pltpu.VMEM((zrows, f_dim), jnp.float32),
            pltpu.SemaphoreType.DMA,
        ],
    )(xp, row, col, w_flat, z)

    out3 = out2.reshape(c_dim, n_nodes, f_dim)
    return out3.transpose(1, 0, 2).reshape(n_nodes, c_dim * f_dim)


# speedup vs baseline: 16.4476x; 1.0100x over previous
import functools

import jax
import jax.numpy as jnp
from jax import lax
from jax.experimental import pallas as pl
from jax.experimental.pallas import tpu as pltpu
from jax.experimental.pallas import tpu_sc as plsc

NC = 2
NS = 16
L = 16
K = 80


def _mm_body(x_ref, wl_ref, o_ref):
    o_ref[...] = lax.dot_general(
        x_ref[...], wl_ref[...], (((1,), (1,)), ((), ())),
        preferred_element_type=jnp.float32)


def _att_body(s_ref, wa_ref, w_ref):
    lg = lax.dot_general(
        wa_ref[...], s_ref[...], (((1,), (1,)), ((), ())),
        preferred_element_type=jnp.float32)
    m = jnp.max(lg, axis=0, keepdims=True)
    ex = jnp.exp(lg - m)
    w_ref[...] = ex / jnp.sum(ex, axis=0, keepdims=True)


def _sc_relu_body(n_edges, f_dim,
                  xp_hbm, row_hbm, col_hbm, s_hbm,
                  ridx_v, cidx_v, a_v, b_v, sem_a, sem_b):
    nw = NC * NS
    epw = n_edges // nw
    nchunks = epw // K
    nj = f_dim // L

    cid = lax.axis_index("c")
    sid = lax.axis_index("s")
    base = (sid * NC + cid) * epw

    def chunk_body(t, carry):
        cb = base + t * K
        pltpu.sync_copy(row_hbm.at[pl.ds(cb, K)], ridx_v)
        pltpu.sync_copy(col_hbm.at[pl.ds(cb, K)], cidx_v)
        cp_a = pltpu.async_copy(xp_hbm.at[ridx_v], a_v, sem_a)
        cp_b = pltpu.async_copy(xp_hbm.at[cidx_v], b_v, sem_b)
        cp_a.wait()
        cp_b.wait()

        def edge_body(e, ecarry):
            for j in range(nj):
                av = a_v[e, pl.ds(j * L, L)]
                bv = b_v[e, pl.ds(j * L, L)]
                a_v[e, pl.ds(j * L, L)] = jnp.maximum(0.5 * av + bv, 0.0)
            return ecarry

        lax.fori_loop(0, K, edge_body, 0)
        pltpu.sync_copy(a_v, s_hbm.at[pl.ds(cb, K)])
        return carry

    lax.fori_loop(0, nchunks, chunk_body, 0)


def _sc_scatter_body(n_nodes, n_edges, c_dim, f_dim,
                     xp_hbm, row_hbm, col_hbm, w_hbm, z_hbm, out_hbm,
                     acc_sh, *scr):
    ridx = scr[0:4]
    cidx = scr[4:8]
    wv = scr[8:12]
    bv = scr[12:14]
    yv = scr[14:16]
    zb_v, ob_v = scr[16], scr[17]
    sem_i = scr[18:22]
    sem_g = scr[22:24]
    sem_s = scr[24:26]

    KS = bv[0].shape[0]
    eps = n_edges // NS
    nch = eps // KS
    zrows = z_hbm.shape[0]
    stripe = zrows * 8
    nj = f_dim // L

    cid = lax.axis_index("c")
    sid = lax.axis_index("s")

    rbase = sid * stripe
    my_rows = jnp.clip(n_nodes - rbase, 0, stripe)
    nz = my_rows // zrows

    pltpu.sync_copy(z_hbm, zb_v)
    dummy_rows = xp_hbm.at[pl.ds(0, KS)]
    dummy_idx = row_hbm.at[pl.ds(0, KS)]

    for cpass in range(c_dim // NC):
        c_val = cid * (c_dim // NC) + cpass
        ebase = sid * eps

        def zero_body(r, carry):
            pltpu.sync_copy(zb_v, acc_sh.at[pl.ds(rbase + r * zrows, zrows)])
            return carry

        lax.fori_loop(0, nz, zero_body, 0)
        plsc.subcore_barrier()

        def fire_idx(t, ring):
            cb = ebase + t * KS
            pltpu.async_copy(row_hbm.at[pl.ds(cb, KS)], ridx[ring], sem_i[ring])
            pltpu.async_copy(col_hbm.at[pl.ds(cb, KS)], cidx[ring], sem_i[ring])
            pltpu.async_copy(w_hbm.at[pl.ds(c_val * n_edges + cb, KS)],
                             wv[ring], sem_i[ring])

        def wait_idx(ring):
            pltpu.make_async_copy(dummy_idx, ridx[ring], sem_i[ring]).wait()
            pltpu.make_async_copy(dummy_idx, cidx[ring], sem_i[ring]).wait()
            pltpu.make_async_copy(dummy_idx, wv[ring], sem_i[ring]).wait()

        fire_idx(0, 0)
        fire_idx(1, 1)
        wait_idx(0)
        pltpu.async_copy(xp_hbm.at[cidx[0]], bv[0], sem_g[0])

        def super_body(p, carry):
            for ph in range(4):
                t = 4 * p + ph
                pg = ph % 2

                @pl.when(jnp.logical_and(t >= 2, t < nch + 2))
                def _wait_scatter():
                    pltpu.make_async_copy(dummy_rows, yv[pg], sem_s[pg]).wait()

                @pl.when(t + 2 < nch)
                def _fire_idx():
                    fire_idx(t + 2, (ph + 2) % 4)

                @pl.when(t + 1 < nch)
                def _next_gather():
                    wait_idx((ph + 1) % 4)
                    pltpu.async_copy(xp_hbm.at[cidx[(ph + 1) % 4]],
                                     bv[(ph + 1) % 2], sem_g[(ph + 1) % 2])

                @pl.when(t < nch)
                def _compute_scatter():
                    pltpu.make_async_copy(dummy_rows, bv[pg], sem_g[pg]).wait()
                    b_v, y_v, w_v = bv[pg], yv[pg], wv[ph]

                    def edge_body(e, ecarry):
                        ws = jnp.broadcast_to(w_v[pl.ds(e, 1)], (L,))
                        for j in range(nj):
                            y_v[e, pl.ds(j * L, L)] = (
                                b_v[e, pl.ds(j * L, L)] * ws)
                        return ecarry

                    lax.fori_loop(0, KS, edge_body, 0, unroll=2)
                    pltpu.async_copy(y_v, acc_sh.at[ridx[ph]],
                                     sem_s[pg], add=True)
            return carry

        lax.fori_loop(0, (nch + 2 + 3) // 4, super_body, 0)
        plsc.subcore_barrier()

        def wb_body(r, carry):
            rb = rbase + r * zrows
            pltpu.sync_copy(acc_sh.at[pl.ds(rb, zrows)], ob_v)
            pltpu.sync_copy(ob_v, out_hbm.at[pl.ds(c_val * n_nodes + rb, zrows)])
            return carry

        lax.fori_loop(0, nz, wb_body, 0)
        plsc.subcore_barrier()


def kernel(x, edge_index, W_lin, W_att):
    n_nodes, in_feat = x.shape
    f_dim = W_lin.shape[0]
    c_dim = W_att.shape[0]
    n_edges = edge_index.shape[1]
    nw = NC * NS
    assert n_edges % (nw * K) == 0 and f_dim % L == 0
    assert c_dim % NC == 0

    row = edge_index[0].astype(jnp.int32)
    col = edge_index[1].astype(jnp.int32)

    xp = pl.pallas_call(
        _mm_body,
        out_shape=jax.ShapeDtypeStruct((n_nodes, f_dim), jnp.float32),
    )(x, W_lin)

    mesh = plsc.VectorSubcoreMesh(core_axis_name="c", subcore_axis_name="s")

    s = pl.kernel(
        functools.partial(_sc_relu_body, n_edges, f_dim),
        out_type=jax.ShapeDtypeStruct((n_edges, f_dim), jnp.float32),
        mesh=mesh,
        scratch_types=[
            pltpu.VMEM((K,), jnp.int32),
            pltpu.VMEM((K,), jnp.int32),
            pltpu.VMEM((K, f_dim), jnp.float32),
            pltpu.VMEM((K, f_dim), jnp.float32),
            pltpu.SemaphoreType.DMA,
            pltpu.SemaphoreType.DMA,
        ],
    )(xp, row, col)

    be = 2560
    assert n_edges % be == 0
    w = pl.pallas_call(
        _att_body,
        grid=(n_edges // be,),
        in_specs=[
            pl.BlockSpec((be, f_dim), lambda i: (i, 0)),
            pl.BlockSpec((c_dim, f_dim), lambda i: (0, 0)),
        ],
        out_specs=pl.BlockSpec((c_dim, be), lambda i: (0, i)),
        out_shape=jax.ShapeDtypeStruct((c_dim, n_edges), jnp.float32),
    )(s, W_att)
    w_flat = w.reshape(c_dim * n_edges)

    zrows = 80
    ks = 40
    assert n_nodes % zrows == 0 and n_nodes <= NS * zrows * 8
    assert n_edges % (NS * ks) == 0
    z = jnp.zeros((zrows, f_dim), jnp.float32)
    out2 = pl.kernel(
        functools.partial(_sc_scatter_body, n_nodes, n_edges, c_dim, f_dim),
        out_type=jax.ShapeDtypeStruct((c_dim * n_nodes, f_dim), jnp.float32),
        mesh=mesh,
        scratch_types=(
            [pltpu.VMEM_SHARED((n_nodes, f_dim), jnp.float32)]
            + [pltpu.VMEM((ks,), jnp.int32) for _ in range(8)]
            + [pltpu.VMEM((ks,), jnp.float32) for _ in range(4)]
            + [pltpu.VMEM((ks, f_dim), jnp.float32) for _ in range(4)]
            + [pltpu.VMEM((zrows, f_dim), jnp.float32) for _ in range(2)]
            + [pltpu.SemaphoreType.DMA for _ in range(8)]
        ),
    )(xp, row, col, w_flat, z)

    out3 = out2.reshape(c_dim, n_nodes, f_dim)
    return out3.transpose(1, 0, 2).reshape(n_nodes, c_dim * f_dim)
